# baseline (speedup 1.0000x reference)
import functools

import jax
import jax.numpy as jnp
from jax import lax
from jax.experimental import pallas as pl
from jax.experimental.pallas import tpu as pltpu
from jax.experimental.pallas import tpu_sc as plsc

N = 10000
E = 320000
D = 128
NL = 3

NC = 2
NS = 16
NW = NC * NS

CHUNK = 128
EPW = E // NW
NCH = 80
NPASS = 2
NCHP = NCH // NPASS
EPW_PAD = NCH * CHUNK
ROWS_PER_TILE = 632
NPAD = NS * ROWS_PER_TILE

_sc_mesh = plsc.VectorSubcoreMesh(core_axis_name="c", subcore_axis_name="s")


@functools.partial(
    pl.kernel,
    out_type=jax.ShapeDtypeStruct((NC, NPAD, D), jnp.float32),
    mesh=_sc_mesh,
    scratch_types=[
        pltpu.VMEM((NCHP, CHUNK), jnp.int32),
        pltpu.VMEM((NCHP, CHUNK), jnp.int32),
        pltpu.VMEM((CHUNK, D), jnp.float32),
        pltpu.VMEM((CHUNK, D), jnp.float32),
        pltpu.VMEM_SHARED((NPAD, D), jnp.float32),
        pltpu.SemaphoreType.DMA,
        pltpu.SemaphoreType.DMA,
        pltpu.SemaphoreType.DMA,
        pltpu.SemaphoreType.DMA,
    ],
)
def _sc_aggregate(h_hbm, src_hbm, dst_hbm, zeros_hbm, out_hbm,
                  src_v, dst_v, rows0, rows1, agg_sh, sem0, sem1, sem2, sem3):
    c = lax.axis_index("c")
    s = lax.axis_index("s")
    wid = c * NS + s

    pltpu.sync_copy(zeros_hbm.at[pl.ds(s * ROWS_PER_TILE, ROWS_PER_TILE)],
                    agg_sh.at[pl.ds(s * ROWS_PER_TILE, ROWS_PER_TILE)])
    plsc.subcore_barrier()

    for p in range(NPASS):
        base = wid * NCH + p * NCHP
        pltpu.sync_copy(src_hbm.at[pl.ds(base, NCHP)], src_v)
        pltpu.sync_copy(dst_hbm.at[pl.ds(base, NCHP)], dst_v)

        pltpu.async_copy(h_hbm.at[src_v.at[0]], rows0, sem0)
        pltpu.async_copy(h_hbm.at[src_v.at[1]], rows1, sem1)

        def body(jj, carry):
            j = jj * 2
            pltpu.make_async_copy(h_hbm.at[src_v.at[j]], rows0, sem0).wait()
            pltpu.async_copy(rows0, agg_sh.at[dst_v.at[j]], sem2, add=True)
            pltpu.make_async_copy(h_hbm.at[src_v.at[j + 1]], rows1,
                                  sem1).wait()
            pltpu.async_copy(rows1, agg_sh.at[dst_v.at[j + 1]], sem3,
                             add=True)
            pltpu.make_async_copy(rows0, agg_sh.at[dst_v.at[j]],
                                  sem2).wait()

            @pl.when(j + 2 < NCHP)
            def _():
                pltpu.async_copy(h_hbm.at[src_v.at[j + 2]], rows0, sem0)

            pltpu.make_async_copy(rows1, agg_sh.at[dst_v.at[j + 1]],
                                  sem3).wait()

            @pl.when(j + 3 < NCHP)
            def _():
                pltpu.async_copy(h_hbm.at[src_v.at[j + 3]], rows1, sem1)

            return carry

        lax.fori_loop(0, NCHP // 2, body, 0)

    plsc.subcore_barrier()
    pltpu.sync_copy(
        agg_sh.at[pl.ds(s * ROWS_PER_TILE, ROWS_PER_TILE)],
        out_hbm.at[c].at[pl.ds(s * ROWS_PER_TILE, ROWS_PER_TILE)])


ROW_BLK = 1000
GRID_A = N // ROW_BLK


def _mlp_body(scale_ref, h_ref, a0_ref, a1_ref, w1_ref, b1_ref, w2_ref,
              b2_ref, z2_ref, stats_ref):
    i = pl.program_id(0)
    z = scale_ref[0, 0] * h_ref[...] + a0_ref[0] + a1_ref[0]
    t = jnp.maximum(
        jnp.dot(z, w1_ref[...], preferred_element_type=jnp.float32)
        + b1_ref[...], 0.0)
    z2 = (jnp.dot(t, w2_ref[...], preferred_element_type=jnp.float32)
          + b2_ref[...])
    z2_ref[...] = z2
    part = jnp.concatenate(
        [jnp.sum(z2, axis=0, keepdims=True),
         jnp.sum(z2 * z2, axis=0, keepdims=True),
         jnp.zeros((6, D), jnp.float32)], axis=0)

    @pl.when(i == 0)
    def _():
        stats_ref[...] = part

    @pl.when(i != 0)
    def _():
        stats_ref[...] += part


def _mlp_call(h, agg, w1, b1, w2, b2, scale):
    return pl.pallas_call(
        _mlp_body,
        grid=(GRID_A,),
        in_specs=[
            pl.BlockSpec(memory_space=pltpu.SMEM),
            pl.BlockSpec((ROW_BLK, D), lambda i: (i, 0)),
            pl.BlockSpec((1, ROW_BLK, D), lambda i: (0, i, 0)),
            pl.BlockSpec((1, ROW_BLK, D), lambda i: (1, i, 0)),
            pl.BlockSpec((D, D), lambda i: (0, 0)),
            pl.BlockSpec((1, D), lambda i: (0, 0)),
            pl.BlockSpec((D, D), lambda i: (0, 0)),
            pl.BlockSpec((1, D), lambda i: (0, 0)),
        ],
        out_specs=[
            pl.BlockSpec((ROW_BLK, D), lambda i: (i, 0)),
            pl.BlockSpec((8, D), lambda i: (0, 0)),
        ],
        out_shape=[
            jax.ShapeDtypeStruct((N, D), jnp.float32),
            jax.ShapeDtypeStruct((8, D), jnp.float32),
        ],
    )(scale, h, agg, agg, w1, b1, w2, b2)


def _bn_body(relu, z2_ref, stats_ref, gamma_ref, beta_ref, out_ref):
    mean = stats_ref[0:1, :] * (1.0 / N)
    var = stats_ref[1:2, :] * (1.0 / N) - mean * mean
    inv = lax.rsqrt(var + 1e-5)
    y = (z2_ref[...] - mean) * (inv * gamma_ref[...]) + beta_ref[...]
    if relu:
        y = jnp.maximum(y, 0.0)
    out_ref[...] = y


def _bn_call(z2, stats, gamma, beta, relu):
    return pl.pallas_call(
        functools.partial(_bn_body, relu),
        grid=(GRID_A,),
        in_specs=[
            pl.BlockSpec((ROW_BLK, D), lambda i: (i, 0)),
            pl.BlockSpec((8, D), lambda i: (0, 0)),
            pl.BlockSpec((1, D), lambda i: (0, 0)),
            pl.BlockSpec((1, D), lambda i: (0, 0)),
        ],
        out_specs=pl.BlockSpec((ROW_BLK, D), lambda i: (i, 0)),
        out_shape=jax.ShapeDtypeStruct((N, D), jnp.float32),
    )(z2, stats, gamma, beta)


def kernel(x, edge_index, W1, b1, W2, b2, eps, gamma, beta):
    src = edge_index[0]
    dst = edge_index[1]
    padn = EPW_PAD - EPW
    pad_src = jnp.broadcast_to(
        (jnp.arange(padn, dtype=jnp.int32) * 37) % N, (NW, padn))
    pad_dst = jnp.broadcast_to(
        N + (jnp.arange(padn, dtype=jnp.int32) % (NPAD - N)), (NW, padn))
    src2 = jnp.concatenate([src.reshape(NW, EPW), pad_src], axis=1)
    src2 = src2.reshape(NW * NCH, CHUNK)
    dst2 = jnp.concatenate([dst.reshape(NW, EPW), pad_dst], axis=1)
    dst2 = dst2.reshape(NW * NCH, CHUNK)
    zeros = jnp.zeros((NPAD, D), jnp.float32)

    h = x
    for l in range(NL):
        agg = _sc_aggregate(h, src2, dst2, zeros)
        scale = jnp.reshape(1.0 + eps[l], (1, 1))
        z2, stats = _mlp_call(h, agg, W1[l], b1[l].reshape(1, D),
                              W2[l], b2[l].reshape(1, D), scale)
        h = _bn_call(z2, stats, gamma[l].reshape(1, D),
                     beta[l].reshape(1, D), relu=(l != NL - 1))
    return h


# speedup vs baseline: 1.2564x; 1.2564x over previous
import functools

import jax
import jax.numpy as jnp
from jax import lax
from jax.experimental import pallas as pl
from jax.experimental.pallas import tpu as pltpu
from jax.experimental.pallas import tpu_sc as plsc

N = 10000
E = 320000
D = 128
NL = 3

NC = 2
NS = 16
NW = NC * NS

CHUNK = 128
PASS = 40
NCHW = 2 * PASS
EPW = E // NW
EPW_PAD = NCHW * CHUNK
ROWS_PER_TILE = 632
NPAD = NS * ROWS_PER_TILE

_sc_mesh = plsc.VectorSubcoreMesh(core_axis_name="c", subcore_axis_name="s")


@functools.partial(
    pl.kernel,
    out_type=jax.ShapeDtypeStruct((NC, NPAD, D), jnp.float32),
    mesh=_sc_mesh,
    scratch_types=[
        pltpu.VMEM((PASS, CHUNK), jnp.int32),
        pltpu.VMEM((PASS, CHUNK), jnp.int32),
        pltpu.VMEM((CHUNK, D), jnp.float32),
        pltpu.VMEM((CHUNK, D), jnp.float32),
        pltpu.VMEM_SHARED((NPAD, D), jnp.float32),
        pltpu.SemaphoreType.DMA,
        pltpu.SemaphoreType.DMA,
    ],
)
def _sc_aggregate(h_hbm, ei_hbm, zeros_hbm, out_hbm,
                  src_v, dst_v, rows0, rows1, agg_sh, sem0, sem1):
    c = lax.axis_index("c")
    s = lax.axis_index("s")
    wid = c * NS + s

    pltpu.sync_copy(zeros_hbm.at[pl.ds(s * ROWS_PER_TILE, ROWS_PER_TILE)],
                    agg_sh.at[pl.ds(s * ROWS_PER_TILE, ROWS_PER_TILE)])
    plsc.subcore_barrier()

    for p in range(2):
        loadoff = wid * NCHW + p * PASS
        pltpu.sync_copy(ei_hbm.at[0].at[pl.ds(loadoff, PASS)], src_v)
        pltpu.sync_copy(ei_hbm.at[1].at[pl.ds(loadoff, PASS)], dst_v)

        pltpu.async_copy(h_hbm.at[src_v.at[0]], rows0, sem0)
        pltpu.async_copy(h_hbm.at[src_v.at[1]], rows1, sem1)

        def body(jj, carry):
            j = jj * 2
            pltpu.make_async_copy(h_hbm.at[src_v.at[j]], rows0, sem0).wait()
            pltpu.sync_copy(rows0, agg_sh.at[dst_v.at[j]], add=True)

            @pl.when(j + 2 < PASS)
            def _():
                pltpu.async_copy(h_hbm.at[src_v.at[j + 2]], rows0, sem0)

            pltpu.make_async_copy(h_hbm.at[src_v.at[j + 1]], rows1,
                                  sem1).wait()
            pltpu.sync_copy(rows1, agg_sh.at[dst_v.at[j + 1]], add=True)

            @pl.when(j + 3 < PASS)
            def _():
                pltpu.async_copy(h_hbm.at[src_v.at[j + 3]], rows1, sem1)

            return carry

        lax.fori_loop(0, PASS // 2, body, 0)

    plsc.subcore_barrier()
    pltpu.sync_copy(
        agg_sh.at[pl.ds(s * ROWS_PER_TILE, ROWS_PER_TILE)],
        out_hbm.at[c].at[pl.ds(s * ROWS_PER_TILE, ROWS_PER_TILE)])


ROW_BLK = 1000
GRID_A = N // ROW_BLK


def _mlp_body(scale_ref, h_ref, a0_ref, a1_ref, w1_ref, b1_ref, w2_ref,
              b2_ref, z2_ref, stats_ref):
    i = pl.program_id(0)
    z = scale_ref[0, 0] * h_ref[...] + a0_ref[0] + a1_ref[0]
    t = jnp.maximum(
        jnp.dot(z, w1_ref[...], preferred_element_type=jnp.float32)
        + b1_ref[...], 0.0)
    z2 = (jnp.dot(t, w2_ref[...], preferred_element_type=jnp.float32)
          + b2_ref[...])
    z2_ref[...] = z2
    part = jnp.concatenate(
        [jnp.sum(z2, axis=0, keepdims=True),
         jnp.sum(z2 * z2, axis=0, keepdims=True),
         jnp.zeros((6, D), jnp.float32)], axis=0)

    @pl.when(i == 0)
    def _():
        stats_ref[...] = part

    @pl.when(i != 0)
    def _():
        stats_ref[...] += part


def _mlp_call(h, agg, w1, b1, w2, b2, scale):
    return pl.pallas_call(
        _mlp_body,
        grid=(GRID_A,),
        in_specs=[
            pl.BlockSpec(memory_space=pltpu.SMEM),
            pl.BlockSpec((ROW_BLK, D), lambda i: (i, 0)),
            pl.BlockSpec((1, ROW_BLK, D), lambda i: (0, i, 0)),
            pl.BlockSpec((1, ROW_BLK, D), lambda i: (1, i, 0)),
            pl.BlockSpec((D, D), lambda i: (0, 0)),
            pl.BlockSpec((1, D), lambda i: (0, 0)),
            pl.BlockSpec((D, D), lambda i: (0, 0)),
            pl.BlockSpec((1, D), lambda i: (0, 0)),
        ],
        out_specs=[
            pl.BlockSpec((ROW_BLK, D), lambda i: (i, 0)),
            pl.BlockSpec((8, D), lambda i: (0, 0)),
        ],
        out_shape=[
            jax.ShapeDtypeStruct((N, D), jnp.float32),
            jax.ShapeDtypeStruct((8, D), jnp.float32),
        ],
    )(scale, h, agg, agg, w1, b1, w2, b2)


def _bn_body(relu, z2_ref, stats_ref, gamma_ref, beta_ref, out_ref):
    mean = stats_ref[0:1, :] * (1.0 / N)
    var = stats_ref[1:2, :] * (1.0 / N) - mean * mean
    inv = lax.rsqrt(var + 1e-5)
    y = (z2_ref[...] - mean) * (inv * gamma_ref[...]) + beta_ref[...]
    if relu:
        y = jnp.maximum(y, 0.0)
    out_ref[...] = y


def _bn_call(z2, stats, gamma, beta, relu):
    return pl.pallas_call(
        functools.partial(_bn_body, relu),
        grid=(GRID_A,),
        in_specs=[
            pl.BlockSpec((ROW_BLK, D), lambda i: (i, 0)),
            pl.BlockSpec((8, D), lambda i: (0, 0)),
            pl.BlockSpec((1, D), lambda i: (0, 0)),
            pl.BlockSpec((1, D), lambda i: (0, 0)),
        ],
        out_specs=pl.BlockSpec((ROW_BLK, D), lambda i: (i, 0)),
        out_shape=jax.ShapeDtypeStruct((N, D), jnp.float32),
    )(z2, stats, gamma, beta)


def kernel(x, edge_index, W1, b1, W2, b2, eps, gamma, beta):
    padn = EPW_PAD - EPW
    pad_src = jnp.broadcast_to(
        (jnp.arange(padn, dtype=jnp.int32) * 37) % N, (NW, padn))
    pad_dst = jnp.broadcast_to(
        N + (jnp.arange(padn, dtype=jnp.int32) % (NPAD - N)), (NW, padn))
    src2 = jnp.concatenate([edge_index[0].reshape(NW, EPW), pad_src], axis=1)
    dst2 = jnp.concatenate([edge_index[1].reshape(NW, EPW), pad_dst], axis=1)
    ei = jnp.stack([src2.reshape(NW * NCHW, CHUNK),
                    dst2.reshape(NW * NCHW, CHUNK)])
    zeros = jnp.zeros((NPAD, D), jnp.float32)

    h = x
    for l in range(NL):
        agg = _sc_aggregate(h, ei, zeros)
        scale = jnp.reshape(1.0 + eps[l], (1, 1))
        z2, stats = _mlp_call(h, agg, W1[l], b1[l].reshape(1, D),
                              W2[l], b2[l].reshape(1, D), scale)
        h = _bn_call(z2, stats, gamma[l].reshape(1, D),
                     beta[l].reshape(1, D), relu=(l != NL - 1))
    return h
